# baseline (speedup 1.0000x reference)
import functools

import jax
import jax.numpy as jnp
import numpy as np
from jax import lax
from jax.experimental import pallas as pl
from jax.experimental.pallas import tpu as pltpu
from jax.experimental.pallas import tpu_sc as plsc

NUM_FIELDS = 26
FIELD_DIM = 100000
TOTAL_ROWS = NUM_FIELDS * FIELD_DIM
EMBED_DIM = 16
BATCH = 16384

NC = 2
NS = 16
NW = NC * NS
BW = BATCH // NW
CB = 64
CPW = BW // CB
IPC = CB * NUM_FIELDS
IG = IPC // 128
GPC = CB // 16
IPW = CPW * IPC
NBUF = 2

_OFFSETS = np.array(
    (0, *np.cumsum([FIELD_DIM] * NUM_FIELDS)[:-1]), dtype=np.int32)


def _fm_body(idx_hbm, emb_hbm, fc_hbm, out_hbm,
             idx_v, rows_v, fc_v, out_v, sem0, sem1):
    wid = lax.axis_index("s") * NC + lax.axis_index("c")
    pltpu.sync_copy(idx_hbm.at[wid], idx_v)
    sems = (sem0, sem1)

    def issue(c, b):
        for g in range(IG):
            pltpu.async_copy(
                emb_hbm.at[idx_v.at[pl.ds(c * IPC + g * 128, 128)]],
                rows_v.at[b, pl.ds(g * 128, 128)], sems[b])
            pltpu.async_copy(
                fc_hbm.at[idx_v.at[pl.ds(c * IPC + g * 128, 128)]],
                fc_v.at[b, pl.ds(g * 128, 128)], sems[b])

    def drain(b):
        pltpu.make_async_copy(
            emb_hbm.at[pl.ds(0, IPC)], rows_v.at[b], sems[b]).wait()
        pltpu.make_async_copy(
            fc_hbm.at[pl.ds(0, IPC)], fc_v.at[b], sems[b]).wait()

    def compute(c, b):
        lane = lax.iota(jnp.int32, 16)
        himask = lane >= 6
        zero = jnp.zeros((16,), jnp.float32)

        def group_body(g, _):
            o = g * 16
            acc = zero
            for r in range(16):
                row = (o + r) * NUM_FIELDS
                v1 = fc_v[b, pl.ds(row, 16)]
                v2 = fc_v[b, pl.ds(row + 10, 16)]
                lin = jnp.sum(v1) + jnp.sum(jnp.where(himask, v2, zero))
                e = rows_v[b, row]
                s = e
                q = e * e
                for f in range(1, NUM_FIELDS):
                    e = rows_v[b, row + f]
                    s = s + e
                    q = q + e * e
                acc = jnp.where(lane == r,
                                lin + 0.5 * jnp.sum(s * s - q), acc)
            out_v[pl.ds(c * CB + o, 16)] = acc
            return 0

        lax.fori_loop(0, GPC, group_body, 0)

    for b in range(NBUF):
        issue(b, b)

    def body(i, _):
        c = i * NBUF
        for b in range(NBUF):
            drain(b)
            compute(c + b, b)

            @pl.when(c + b + NBUF < CPW)
            def _():
                issue(c + b + NBUF, b)
        return 0

    lax.fori_loop(0, CPW // NBUF, body, 0)
    pltpu.sync_copy(out_v, out_hbm.at[pl.ds(wid * BW, BW)])


@jax.jit
def _fm(idx, emb_table, fc_flat):
    mesh = plsc.VectorSubcoreMesh(
        core_axis_name="c", subcore_axis_name="s",
        num_cores=NC, num_subcores=NS)
    f = functools.partial(
        pl.kernel,
        out_type=jax.ShapeDtypeStruct((BATCH,), jnp.float32),
        mesh=mesh,
        compiler_params=pltpu.CompilerParams(
            needs_layout_passes=False, use_tc_tiling_on_sc=False),
        scratch_types=[
            pltpu.VMEM((IPW,), jnp.int32),
            pltpu.VMEM((NBUF, IPC, EMBED_DIM), jnp.float32),
            pltpu.VMEM((NBUF, IPC), jnp.float32),
            pltpu.VMEM((BW,), jnp.float32),
            pltpu.SemaphoreType.DMA,
            pltpu.SemaphoreType.DMA,
        ],
    )(_fm_body)
    return f(idx, emb_table, fc_flat)


def kernel(x, emb_table, fc_table, bias):
    idx = x.astype(jnp.int32) + jnp.asarray(_OFFSETS)[None, :]
    idx = idx.reshape(NW, IPW)
    out = _fm(idx, emb_table, fc_table[:, 0])
    return out[:, None] + bias[None, :]
